# Initial kernel scaffold; baseline (speedup 1.0000x reference)
import jax
import jax.numpy as jnp
from jax.experimental import pallas as pl


def kernel(score1, score2, homo12):
    raise NotImplementedError("write your pallas kernel here")


# speedup vs baseline: 1.2751x; 1.2751x over previous
import jax
import jax.numpy as jnp
import numpy as np
from jax.experimental import pallas as pl

_H = 512
_W = 512
_BORDER = 8
_K = 512
_GK = 15
_SIGMA = 0.5
_NEG = -1e30


def _border_mask():
    r = jax.lax.broadcasted_iota(jnp.int32, (_H, _W), 0)
    c = jax.lax.broadcasted_iota(jnp.int32, (_H, _W), 1)
    return ((r >= _BORDER) & (r < _H - _BORDER)
            & (c >= _BORDER) & (c < _W - _BORDER)).astype(jnp.float32)


def _shift_rows(x, d, fill):
    pad = jnp.full((abs(d), _W), fill, x.dtype)
    if d > 0:
        return jnp.concatenate([pad, x[:-d, :]], axis=0)
    return jnp.concatenate([x[-d:, :], pad], axis=0)


def _shift_cols(x, d, fill):
    pad = jnp.full((_H, abs(d)), fill, x.dtype)
    if d > 0:
        return jnp.concatenate([pad, x[:, :-d]], axis=1)
    return jnp.concatenate([x[:, -d:], pad], axis=1)


def _nms_one(x, bm):
    x = x * bm
    x = jnp.where(x < 0.0, 0.0, x)
    mv = x
    for d in (-2, -1, 1, 2):
        mv = jnp.maximum(mv, _shift_rows(x, d, _NEG))
    mh = mv
    for d in (-2, -1, 1, 2):
        mh = jnp.maximum(mh, _shift_cols(mv, d, _NEG))
    keep = (x >= mh) & (x > 0.0)
    return x * keep.astype(x.dtype)


def _nms_kernel(s1_ref, w2_ref, o1_ref, o2_ref):
    bm = _border_mask()
    o1_ref[0] = _nms_one(s1_ref[0], bm)
    o2_ref[0] = _nms_one(w2_ref[0], bm)


def _loss_kernel(s1_ref, g_ref, t_ref, v_ref, ls_ref, vs_ref):
    mu = _GK // 2
    wts = np.exp(-((np.arange(_GK) - mu) ** 2) / (2.0 * _SIGMA ** 2)).astype(np.float32)
    x = g_ref[0] * t_ref[0]
    acc = wts[mu] * x
    for d in range(1, mu + 1):
        acc = acc + wts[mu - d] * _shift_rows(x, d, 0.0)
        acc = acc + wts[mu + d] * _shift_rows(x, -d, 0.0)
    g = wts[mu] * acc
    for d in range(1, mu + 1):
        g = g + wts[mu - d] * _shift_cols(acc, d, 0.0)
        g = g + wts[mu + d] * _shift_cols(acc, -d, 0.0)
    s1f = s1_ref[0] * _border_mask()
    v = v_ref[0]
    diff = s1f - g
    i = pl.program_id(0)
    ls_ref[pl.ds(i, 1), :] = jnp.sum(diff * diff * v).reshape(1, 1)
    vs_ref[pl.ds(i, 1), :] = jnp.sum(v).reshape(1, 1)


def _warp(img, homo):
    B, C, H, W = img.shape
    ys, xs = jnp.meshgrid(jnp.arange(H, dtype=jnp.float32),
                          jnp.arange(W, dtype=jnp.float32), indexing='ij')
    grid = jnp.stack([xs.reshape(-1), ys.reshape(-1),
                      jnp.ones(H * W, jnp.float32)], axis=0)

    def one(im_b, h_b):
        w = h_b @ grid
        x = w[0] / (w[2] + 1e-8)
        y = w[1] / (w[2] + 1e-8)
        x0 = jnp.floor(x); y0 = jnp.floor(y)
        x1 = x0 + 1.0; y1 = y0 + 1.0
        wa = (x1 - x) * (y1 - y)
        wb = (x1 - x) * (y - y0)
        wc = (x - x0) * (y1 - y)
        wd = (x - x0) * (y - y0)

        def gather(xi, yi):
            xi_c = jnp.clip(xi, 0.0, W - 1.0).astype(jnp.int32)
            yi_c = jnp.clip(yi, 0.0, H - 1.0).astype(jnp.int32)
            valid = ((xi >= 0.0) & (xi <= W - 1.0)
                     & (yi >= 0.0) & (yi <= H - 1.0)).astype(im_b.dtype)
            return im_b[:, yi_c, xi_c] * valid[None, :]

        out = (wa[None] * gather(x0, y0) + wb[None] * gather(x0, y1)
               + wc[None] * gather(x1, y0) + wd[None] * gather(x1, y1))
        return out.reshape(C, H, W)

    return jax.vmap(one)(img, homo)


def kernel(score1, score2, homo12):
    B, C, H, W = score1.shape
    w_score2 = _warp(score2, homo12)
    w_vis = (_warp(jnp.ones_like(score2), homo12) > 0).astype(jnp.float32)

    s1 = score1.reshape(B, H, W)
    w2 = w_score2.reshape(B, H, W)

    o1, o2 = pl.pallas_call(
        _nms_kernel,
        grid=(B,),
        in_specs=[pl.BlockSpec((1, H, W), lambda i: (i, 0, 0)),
                  pl.BlockSpec((1, H, W), lambda i: (i, 0, 0))],
        out_specs=[pl.BlockSpec((1, H, W), lambda i: (i, 0, 0)),
                   pl.BlockSpec((1, H, W), lambda i: (i, 0, 0))],
        out_shape=[jax.ShapeDtypeStruct((B, H, W), jnp.float32),
                   jax.ShapeDtypeStruct((B, H, W), jnp.float32)],
    )(s1, w2)

    _, idx1 = jax.lax.top_k(o1.reshape(B, H * W), _K)
    kp = jnp.stack([idx1 // W, idx1 % W], axis=-1).reshape(B, C, _K, 2)

    _, idx2 = jax.lax.top_k(o2.reshape(B, H * W), _K)
    rows = jnp.repeat(jnp.arange(B), _K)
    tmask = jnp.zeros((B, H * W), jnp.float32).at[rows, idx2.reshape(-1)].set(1.0)
    tmask = tmask.reshape(B, H, W)

    ls, vs = pl.pallas_call(
        _loss_kernel,
        grid=(B,),
        in_specs=[pl.BlockSpec((1, H, W), lambda i: (i, 0, 0)),
                  pl.BlockSpec((1, H, W), lambda i: (i, 0, 0)),
                  pl.BlockSpec((1, H, W), lambda i: (i, 0, 0)),
                  pl.BlockSpec((1, H, W), lambda i: (i, 0, 0))],
        out_specs=[pl.BlockSpec((4, 1), lambda i: (0, 0)),
                   pl.BlockSpec((4, 1), lambda i: (0, 0))],
        out_shape=[jax.ShapeDtypeStruct((B, 1), jnp.float32),
                   jax.ShapeDtypeStruct((B, 1), jnp.float32)],
    )(s1, o2, tmask, w_vis.reshape(B, H, W))

    loss = ls.sum() / vs.sum() * 100.0
    return (loss, kp)
